# Initial kernel scaffold; baseline (speedup 1.0000x reference)
import jax
import jax.numpy as jnp
from jax.experimental import pallas as pl


def kernel(values, stm_indices, nstm_indices, ft_w, ft_b, fft_w, fft_b, out_w, out_b):
    raise NotImplementedError("write your pallas kernel here")


# speedup vs baseline: 19.1079x; 19.1079x over previous
import functools

import jax
import jax.numpy as jnp
from jax import lax
from jax.experimental import pallas as pl
from jax.experimental.pallas import tpu as pltpu
from jax.experimental.pallas import tpu_sc as plsc

FT_OUT = 512
MAX_F = 32
N_FT = 49152
N_FFT = 768
B = 4096

NC = 2
NS = 16
NW = NC * NS
R = B // NW
NO = FT_OUT // 16

_PVEC = FT_OUT + 2 * FT_OUT + 16


def _combine_body(ft_ref, fft_ref, o_ref):
    o_ref[...] = ft_ref[...] + fft_ref[...]


def _build_w2(ft_w, fft_w):
    return pl.pallas_call(
        _combine_body,
        grid=(N_FT // N_FFT,),
        in_specs=[
            pl.BlockSpec((N_FFT, FT_OUT), lambda i: (i, 0)),
            pl.BlockSpec((N_FFT, FT_OUT), lambda i: (0, 0)),
        ],
        out_specs=pl.BlockSpec((N_FFT, FT_OUT), lambda i: (i, 0)),
        out_shape=jax.ShapeDtypeStruct((N_FT, FT_OUT), jnp.float32),
    )(ft_w, fft_w)


def _sc_body(w2, vals_flat, stm, nstm, pvec, out_hbm,
             stm_v, nstm_v, vals_v, pvec_v,
             gs0, gn0, gs1, gn1, tbuf, logit_v,
             sem0, sem1):
    wid = lax.axis_index("s") * NC + lax.axis_index("c")
    base = wid * R

    pltpu.sync_copy(stm.at[pl.ds(base, R), :], stm_v)
    pltpu.sync_copy(nstm.at[pl.ds(base, R), :], nstm_v)
    pltpu.sync_copy(vals_flat.at[pl.ds(base * MAX_F, R * MAX_F)], vals_v)
    pltpu.sync_copy(pvec, pvec_v)

    def issue(r, gs, gn, sem):
        pltpu.async_copy(w2.at[stm_v.at[r]], gs, sem)
        pltpu.async_copy(w2.at[nstm_v.at[r]], gn, sem)

    def drain(gs, gn, sem):
        pltpu.make_async_copy(w2.at[pl.ds(0, MAX_F)], gs, sem).wait()
        pltpu.make_async_copy(w2.at[pl.ds(0, MAX_F)], gn, sem).wait()

    def compute(r, gs, gn):
        wv0 = vals_v[pl.ds(r * MAX_F, 16)]
        wv1 = vals_v[pl.ds(r * MAX_F + 16, 16)]
        ws = [wv0[f] for f in range(16)] + [wv1[f] for f in range(16)]

        def obody(o, t):
            o16 = o * 16
            sl = pl.ds(o16, 16)
            a_s = pvec_v[sl]
            a_n = a_s
            for f in range(MAX_F):
                a_s = a_s + ws[f] * gs[f, sl]
                a_n = a_n + ws[f] * gn[f, sl]
            hs = jnp.clip(a_s, 0.0, 1.0)
            hn = jnp.clip(a_n, 0.0, 1.0)
            return (t + hs * pvec_v[pl.ds(FT_OUT + o16, 16)]
                      + hn * pvec_v[pl.ds(2 * FT_OUT + o16, 16)])

        t = lax.fori_loop(0, NO, obody, jnp.zeros((16,), jnp.float32))
        tbuf[pl.ds(r * 16, 16)] = t

    issue(0, gs0, gn0, sem0)

    def body(i, carry):
        r0 = 2 * i
        issue(r0 + 1, gs1, gn1, sem1)
        drain(gs0, gn0, sem0)
        compute(r0, gs0, gn0)

        @pl.when(r0 + 2 < R)
        def _():
            issue(r0 + 2, gs0, gn0, sem0)

        drain(gs1, gn1, sem1)
        compute(r0 + 1, gs1, gn1)
        return carry

    lax.fori_loop(0, R // 2, body, 0)

    ob = pvec_v[pl.ds(3 * FT_OUT, 16)]
    lanes16 = lax.iota(jnp.int32, 16) * 16
    for j in range(R // 16):
        s = ob
        for p in range(16):
            idx = lanes16 + (j * 256 + p)
            s = s + plsc.load_gather(tbuf, [idx])
        logit_v[pl.ds(j * 16, 16)] = 1.0 / (1.0 + jnp.exp(-s))
    pltpu.sync_copy(logit_v, out_hbm.at[pl.ds(base, R)])


@jax.jit
def _sc_bag(w2, vals_flat, stm, nstm, pvec):
    mesh = plsc.VectorSubcoreMesh(core_axis_name="c", subcore_axis_name="s")
    f = pl.kernel(
        _sc_body,
        mesh=mesh,
        out_type=jax.ShapeDtypeStruct((B,), jnp.float32),
        compiler_params=pltpu.CompilerParams(needs_layout_passes=False),
        scratch_types=[
            pltpu.VMEM((R, MAX_F), jnp.int32),
            pltpu.VMEM((R, MAX_F), jnp.int32),
            pltpu.VMEM((R * MAX_F,), jnp.float32),
            pltpu.VMEM((_PVEC,), jnp.float32),
            pltpu.VMEM((MAX_F, FT_OUT), jnp.float32),
            pltpu.VMEM((MAX_F, FT_OUT), jnp.float32),
            pltpu.VMEM((MAX_F, FT_OUT), jnp.float32),
            pltpu.VMEM((MAX_F, FT_OUT), jnp.float32),
            pltpu.VMEM((R * 16,), jnp.float32),
            pltpu.VMEM((R,), jnp.float32),
            pltpu.SemaphoreType.DMA,
            pltpu.SemaphoreType.DMA,
        ],
    )
    return f(w2, vals_flat, stm, nstm, pvec)


def kernel(values, stm_indices, nstm_indices, ft_w, ft_b, fft_w, fft_b,
           out_w, out_b):
    w2 = _build_w2(ft_w, fft_w)
    pvec = jnp.concatenate([
        ft_b + fft_b,
        out_w.reshape(-1),
        jnp.broadcast_to(out_b, (16,)),
    ]).astype(jnp.float32)
    out = _sc_bag(w2, values.reshape(-1),
                  stm_indices.astype(jnp.int32),
                  nstm_indices.astype(jnp.int32), pvec)
    return out.reshape(B, 1)
